# Initial kernel scaffold; baseline (speedup 1.0000x reference)
import jax
import jax.numpy as jnp
from jax.experimental import pallas as pl


def kernel(x, edge_index, W_l1, b_l1, W_r1, g1, be1, rm1, rv1, W_l2, b_l2, W_r2, g2, be2, rm2, rv2, W_l3, b_l3, W_r3):
    raise NotImplementedError("write your pallas kernel here")


# speedup vs baseline: 3.5295x; 3.5295x over previous
import functools

import jax
import jax.numpy as jnp
from jax import lax
from jax.experimental import pallas as pl
from jax.experimental.pallas import tpu as pltpu
from jax.experimental.pallas import tpu_sc as plsc

_N = 10000
_E = 320000
_ROWS = 2528
_NACC = 10240
_BN = 400

_mesh = plsc.VectorSubcoreMesh(core_axis_name="c", subcore_axis_name="s")


def _fill(ref, rows, val):
    v = jnp.full((16,), val, jnp.float32)
    for i in range(rows):
        for k in range(8):
            ref[i, pl.ds(k * 16, 16)] = v


def _zero_acc(zb, acc_s, s):
    def zloop(j, carry):
        pltpu.sync_copy(zb, acc_s.at[pl.ds(s * 640 + j * 16, 16)])
        return carry

    lax.fori_loop(0, 40, zloop, 0)


def _drain_acc(acc_s, out, rows_v, c, s):
    def oloop(k, carry):
        ob = s * 640 + k * 128
        pltpu.sync_copy(acc_s.at[pl.ds(ob, 128)], rows_v)
        pltpu.sync_copy(rows_v, out.at[c, pl.ds(ob, 128)])
        return carry

    lax.fori_loop(0, 5, oloop, 0)


@functools.partial(
    pl.kernel,
    mesh=_mesh,
    out_type=[
        jax.ShapeDtypeStruct((2, _NACC, 128), jnp.float32),
        jax.ShapeDtypeStruct((2, _NACC, 128), jnp.float32),
    ],
    scratch_types=[
        pltpu.VMEM((1, 128), jnp.int32),
        pltpu.VMEM((1, 128), jnp.int32),
        pltpu.VMEM((128, 128), jnp.float32),
        pltpu.VMEM((128, 128), jnp.float32),
        pltpu.VMEM((16, 128), jnp.float32),
        pltpu.VMEM_SHARED((_NACC, 128), jnp.float32),
        pltpu.SemaphoreType.DMA,
    ],
)
def _sage_l1(x_hbm, src_hbm, dst_hbm, acc_out, deg_out,
             srcb, dstb, rows_v, ones_v, zb, acc_s, sem):
    c = lax.axis_index("c")
    s = lax.axis_index("s")
    _fill(zb, 16, 0.0)
    _fill(ones_v, 128, 1.0)

    base_row = (c * 16 + s) * (_ROWS // 32)

    _zero_acc(zb, acc_s, s)
    plsc.subcore_barrier()

    def dbody(j, carry):
        pltpu.sync_copy(dst_hbm.at[base_row + j], dstb)
        pltpu.sync_copy(ones_v, acc_s.at[dstb.at[0]], add=True)
        return carry

    lax.fori_loop(0, _ROWS // 32, dbody, 0)
    plsc.subcore_barrier()
    _drain_acc(acc_s, deg_out, rows_v, c, s)
    _zero_acc(zb, acc_s, s)
    plsc.subcore_barrier()

    def body(j, carry):
        row = base_row + j
        pltpu.sync_copy(src_hbm.at[0, row], srcb)
        pltpu.sync_copy(dst_hbm.at[row], dstb)
        pltpu.async_copy(x_hbm.at[srcb.at[0]], rows_v, sem).wait()
        pltpu.sync_copy(rows_v, acc_s.at[dstb.at[0]], add=True)
        return carry

    lax.fori_loop(0, _ROWS // 32, body, 0)
    plsc.subcore_barrier()
    _drain_acc(acc_s, acc_out, rows_v, c, s)


@functools.partial(
    pl.kernel,
    mesh=_mesh,
    out_type=[
        jax.ShapeDtypeStruct((2, _NACC, 128), jnp.float32),
    ],
    scratch_types=[
        pltpu.VMEM((1, 128), jnp.int32),
        pltpu.VMEM((1, 128), jnp.int32),
        pltpu.VMEM((128, 128), jnp.float32),
        pltpu.VMEM((16, 128), jnp.float32),
        pltpu.VMEM_SHARED((_NACC, 128), jnp.float32),
        pltpu.SemaphoreType.DMA,
    ],
)
def _sage_l23(h_hbm, src_hbm, dst_hbm, acc_out,
              srcb, dstb, rows_v, zb, acc_s, sem):
    c = lax.axis_index("c")
    s = lax.axis_index("s")
    _fill(zb, 16, 0.0)
    _zero_acc(zb, acc_s, s)
    plsc.subcore_barrier()

    base_row = s * (_ROWS // 16)

    def body(j, carry):
        row = base_row + j
        pltpu.sync_copy(src_hbm.at[c, row], srcb)
        pltpu.sync_copy(dst_hbm.at[row], dstb)
        pltpu.async_copy(h_hbm.at[srcb.at[0]], rows_v, sem).wait()
        pltpu.sync_copy(rows_v, acc_s.at[dstb.at[0]], add=True)
        return carry

    lax.fori_loop(0, _ROWS // 16, body, 0)
    plsc.subcore_barrier()
    _drain_acc(acc_s, acc_out, rows_v, c, s)


def _tc1_body(acc_ref, deg_ref, x_ref, wl_ref, wr_ref, b_ref, out_ref):
    deg = deg_ref[0][:, :1] + deg_ref[1][:, :1]
    inv = 1.0 / jnp.maximum(deg, 1.0)
    agg = (acc_ref[0] + acc_ref[1]) * inv
    h = (jnp.dot(agg, wl_ref[...], preferred_element_type=jnp.float32)
         + jnp.dot(x_ref[...], wr_ref[...], preferred_element_type=jnp.float32)
         + b_ref[...])
    h = jnp.maximum(h, 0.0)
    out_ref[0] = h[:, :128]
    out_ref[1] = h[:, 128:]


def _tc23_pre(acc_ref, deg_ref, h_ref, wl_ref, wr_ref, b_ref):
    deg = deg_ref[0][:, :1] + deg_ref[1][:, :1]
    inv = 1.0 / jnp.maximum(deg, 1.0)
    return (jnp.dot(acc_ref[0] * inv, wl_ref[0], preferred_element_type=jnp.float32)
            + jnp.dot(acc_ref[1] * inv, wl_ref[1], preferred_element_type=jnp.float32)
            + jnp.dot(h_ref[0], wr_ref[0], preferred_element_type=jnp.float32)
            + jnp.dot(h_ref[1], wr_ref[1], preferred_element_type=jnp.float32)
            + b_ref[...])


def _tc2_body(acc_ref, deg_ref, h_ref, wl_ref, wr_ref, b_ref, out_ref):
    h = jnp.maximum(_tc23_pre(acc_ref, deg_ref, h_ref, wl_ref, wr_ref, b_ref), 0.0)
    out_ref[0] = h[:, :128]
    out_ref[1] = h[:, 128:]


def _tc3_body(acc_ref, deg_ref, h_ref, wl_ref, wr_ref, b_ref, out_ref):
    pre = _tc23_pre(acc_ref, deg_ref, h_ref, wl_ref, wr_ref, b_ref)
    m = jnp.max(pre, axis=1, keepdims=True)
    e = jnp.exp(pre - m)
    lse = jnp.log(jnp.sum(e, axis=1, keepdims=True))
    out_ref[...] = pre - m - lse


_halves = pl.BlockSpec((2, _BN, 128), lambda i: (0, i, 0))


def _full(shape):
    return pl.BlockSpec(shape, lambda i: tuple(0 for _ in shape))


_tc1_call = pl.pallas_call(
    _tc1_body,
    grid=(_N // _BN,),
    in_specs=[_halves, _halves, pl.BlockSpec((_BN, 128), lambda i: (i, 0)),
              _full((128, 256)), _full((128, 256)), _full((1, 256))],
    out_specs=_halves,
    out_shape=jax.ShapeDtypeStruct((2, _N, 128), jnp.float32),
)

_tc2_call = pl.pallas_call(
    _tc2_body,
    grid=(_N // _BN,),
    in_specs=[_halves, _halves, _halves,
              _full((2, 128, 256)), _full((2, 128, 256)), _full((1, 256))],
    out_specs=_halves,
    out_shape=jax.ShapeDtypeStruct((2, _N, 128), jnp.float32),
)

_tc3_call = pl.pallas_call(
    _tc3_body,
    grid=(_N // _BN,),
    in_specs=[_halves, _halves, _halves,
              _full((2, 128, 40)), _full((2, 128, 40)), _full((1, 40))],
    out_specs=pl.BlockSpec((_BN, 40), lambda i: (i, 0)),
    out_shape=jax.ShapeDtypeStruct((_N, 40), jnp.float32),
)


def kernel(x, edge_index, W_l1, b_l1, W_r1, g1, be1, rm1, rv1,
           W_l2, b_l2, W_r2, g2, be2, rm2, rv2, W_l3, b_l3, W_r3):
    src = edge_index[0]
    dst = edge_index[1]
    pad = _ROWS * 128 - _E
    srcp = jnp.concatenate([src, jnp.zeros((pad,), jnp.int32)])
    dstp = jnp.concatenate([dst, jnp.full((pad,), _N, jnp.int32)])
    src2 = jnp.stack([srcp, srcp + _N]).reshape(2, _ROWS, 1, 128)
    dst2 = dstp.reshape(_ROWS, 1, 128)

    s1 = g1 * lax.rsqrt(rv1 + 1e-5)
    wl1 = W_l1.T * s1
    wr1 = W_r1.T * s1
    bb1 = ((b_l1 - rm1) * s1 + be1).reshape(1, 256)
    s2 = g2 * lax.rsqrt(rv2 + 1e-5)
    wl2 = (W_l2.T * s2).reshape(2, 128, 256)
    wr2 = (W_r2.T * s2).reshape(2, 128, 256)
    bb2 = ((b_l2 - rm2) * s2 + be2).reshape(1, 256)
    wl3 = W_l3.T.reshape(2, 128, 40)
    wr3 = W_r3.T.reshape(2, 128, 40)
    bb3 = b_l3.reshape(1, 40)

    acc1, deg = _sage_l1(x, src2, dst2)
    h1 = _tc1_call(acc1, deg, x, wl1, wr1, bb1)
    (acc2,) = _sage_l23(h1.reshape(2 * _N, 128), src2, dst2)
    h2 = _tc2_call(acc2, deg, h1, wl2, wr2, bb2)
    (acc3,) = _sage_l23(h2.reshape(2 * _N, 128), src2, dst2)
    return _tc3_call(acc3, deg, h2, wl3, wr3, bb3)
